# baseline (speedup 1.0000x reference)
import functools

import jax
import jax.numpy as jnp
from jax import lax
from jax.experimental import pallas as pl
from jax.experimental.pallas import tpu as pltpu
from jax.experimental.pallas import tpu_sc as plsc

N = 10000
E = 320000
D = 128
H = 64
OUT = 128

NC = 2
NS = 16
LANES = 128
REAL_ROWS = E // LANES
ROWS = 2560
RPT = ROWS // (NC * NS)
TAIL = REAL_ROWS - RPT * (NC * NS - 1)
RPS = N // NS
CHUNK = 125
GROUP = 2
DR = 2
DROWS = ROWS // DR
RPT_D = RPT // DR
TAIL_D = TAIL // DR


_R = 2000


def _tc1_body(x_ref, wl_ref, wr_ref, z_ref, r_ref):
    xb = x_ref[...]
    z_ref[...] = jnp.dot(xb, wl_ref[...], preferred_element_type=jnp.float32)
    r_ref[...] = jnp.dot(xb, wr_ref[...], preferred_element_type=jnp.float32)


def _tc1(x, W1l, W1r):
    return pl.pallas_call(
        _tc1_body,
        grid=(N // _R,),
        in_specs=[
            pl.BlockSpec((_R, D), lambda i: (i, 0)),
            pl.BlockSpec((D, H), lambda i: (0, 0)),
            pl.BlockSpec((D, H), lambda i: (0, 0)),
        ],
        out_specs=[
            pl.BlockSpec((_R, H), lambda i: (i, 0)),
            pl.BlockSpec((_R, H), lambda i: (i, 0)),
        ],
        out_shape=[
            jax.ShapeDtypeStruct((N, H), jnp.float32),
            jax.ShapeDtypeStruct((N, H), jnp.float32),
        ],
    )(x, W1l, W1r)


def _elu(v):
    return jnp.where(v > 0, v, jnp.exp(jnp.minimum(v, 0.0)) - 1.0)


def _tc2_body(p_ref, c_ref, r_ref, b_ref, h_ref):
    p = p_ref[...]
    agg = p[0] + p[1]
    c = c_ref[...]
    cnt = c[0, :, 0] + c[1, :, 0]
    inv = 1.0 / jnp.maximum(cnt, 1.0)
    pre = agg * inv[:, None] + b_ref[...] + r_ref[...]
    h_ref[...] = _elu(pre)


def _tc2(P, C, r1, b1):
    return pl.pallas_call(
        _tc2_body,
        grid=(N // _R,),
        in_specs=[
            pl.BlockSpec((NC, _R, H), lambda i: (0, i, 0)),
            pl.BlockSpec((NC, _R, 16), lambda i: (0, i, 0)),
            pl.BlockSpec((_R, H), lambda i: (i, 0)),
            pl.BlockSpec((1, H), lambda i: (0, 0)),
        ],
        out_specs=pl.BlockSpec((_R, H), lambda i: (i, 0)),
        out_shape=jax.ShapeDtypeStruct((N, H), jnp.float32),
    )(P, C, r1, b1.reshape(1, H))


def _tc3_body(p_ref, c_ref, h1_ref, w2l_ref, b2_ref, w2r_ref, w3_ref, b3_ref,
              o_ref):
    p = p_ref[...]
    agg = p[0] + p[1]
    c = c_ref[...]
    cnt = c[0, :, 0] + c[1, :, 0]
    inv = 1.0 / jnp.maximum(cnt, 1.0)
    mean2 = agg * inv[:, None]
    h1 = h1_ref[...]
    pre = (jnp.dot(mean2, w2l_ref[...], preferred_element_type=jnp.float32)
           + b2_ref[...]
           + jnp.dot(h1, w2r_ref[...], preferred_element_type=jnp.float32))
    h2 = _elu(pre)
    o_ref[...] = (jnp.dot(h2, w3_ref[...], preferred_element_type=jnp.float32)
                  + b3_ref[...])


def _tc3(P, C, h1, W2l, b2, W2r, W3, b3):
    return pl.pallas_call(
        _tc3_body,
        grid=(N // _R,),
        in_specs=[
            pl.BlockSpec((NC, _R, H), lambda i: (0, i, 0)),
            pl.BlockSpec((NC, _R, 16), lambda i: (0, i, 0)),
            pl.BlockSpec((_R, H), lambda i: (i, 0)),
            pl.BlockSpec((H, OUT), lambda i: (0, 0)),
            pl.BlockSpec((1, OUT), lambda i: (0, 0)),
            pl.BlockSpec((H, OUT), lambda i: (0, 0)),
            pl.BlockSpec((OUT, OUT), lambda i: (0, 0)),
            pl.BlockSpec((1, OUT), lambda i: (0, 0)),
        ],
        out_specs=pl.BlockSpec((_R, OUT), lambda i: (i, 0)),
        out_shape=jax.ShapeDtypeStruct((N, OUT), jnp.float32),
    )(P, C, h1, W2l, b2.reshape(1, OUT), W2r, W3, b3.reshape(1, OUT))


def _sc_body(with_counts, *refs):
    if with_counts:
        (z_hbm, src_hbm, dst_hbm, out_hbm, cnt_hbm,
         src_v, dst_v, rows_v, zbuf, agg_sh, gsems, ssems, csems,
         ones_v, cbuf, cnt_sh) = refs
    else:
        (z_hbm, src_hbm, dst_hbm, out_hbm,
         src_v, dst_v, rows_v, zbuf, agg_sh, gsems, ssems) = refs

    cid = lax.axis_index("c")
    sid = lax.axis_index("s")
    wid = cid * NS + sid

    zvec = jnp.zeros((16,), jnp.float32)

    def zb(i, carry):
        zbuf[i // 4, pl.ds((i % 4) * 16, 16)] = zvec
        return carry
    lax.fori_loop(0, CHUNK * 4, zb, 0)
    def zcp(k, carry):
        pltpu.sync_copy(zbuf, agg_sh.at[pl.ds(sid * RPS + k * CHUNK, CHUNK)])
        return carry
    lax.fori_loop(0, RPS // CHUNK, zcp, 0)

    if with_counts:
        def cb(i, carry):
            cbuf[i, :] = zvec
            return carry
        lax.fori_loop(0, CHUNK, cb, 0)
        def ccp(k, carry):
            pltpu.sync_copy(cbuf, cnt_sh.at[pl.ds(sid * RPS + k * CHUNK, CHUNK)])
            return carry
        lax.fori_loop(0, RPS // CHUNK, ccp, 0)
        lane = lax.iota(jnp.int32, 16)
        onevec = jnp.where(lane == 0, 1.0, 0.0).astype(jnp.float32)
        def ob(i, carry):
            ones_v[i, :] = onevec
            return carry
        lax.fori_loop(0, DR * LANES, ob, 0)

    pltpu.sync_copy(src_hbm.at[pl.ds(wid * RPT_D, RPT_D)], src_v)
    pltpu.sync_copy(dst_hbm.at[pl.ds(wid * RPT_D, RPT_D)], dst_v)

    plsc.subcore_barrier()

    def make_step(group):
        def step(i, carry):
            j0 = group * i
            gd, sd, cd = [], [], []
            for k in range(group):
                gd.append(pltpu.async_copy(
                    z_hbm.at[src_v.at[j0 + k]], rows_v.at[k], gsems.at[k]))
            for k in range(group):
                gd[k].wait()
                sd.append(pltpu.async_copy(
                    rows_v.at[k], agg_sh.at[dst_v.at[j0 + k]], ssems.at[k],
                    add=True))
                if with_counts:
                    cd.append(pltpu.async_copy(
                        ones_v, cnt_sh.at[dst_v.at[j0 + k]], csems.at[k],
                        add=True))
            for d in sd + cd:
                d.wait()
            return carry
        return step

    @pl.when(wid == NC * NS - 1)
    def _():
        lax.fori_loop(0, TAIL_D // GROUP, make_step(GROUP), 0)

    @pl.when(wid != NC * NS - 1)
    def _():
        lax.fori_loop(0, RPT_D // GROUP, make_step(GROUP), 0)

    plsc.subcore_barrier()

    def ocp(k, carry):
        base = sid * RPS + k * CHUNK
        pltpu.sync_copy(agg_sh.at[pl.ds(base, CHUNK)], zbuf)
        pltpu.sync_copy(zbuf, out_hbm.at[cid, pl.ds(base, CHUNK)])
        if with_counts:
            pltpu.sync_copy(cnt_sh.at[pl.ds(base, CHUNK)], cbuf)
            pltpu.sync_copy(cbuf, cnt_hbm.at[cid, pl.ds(base, CHUNK)])
        return carry
    lax.fori_loop(0, RPS // CHUNK, ocp, 0)


def _sc_agg(z, src2d, dst2d, with_counts):
    mesh = plsc.VectorSubcoreMesh(
        core_axis_name="c", subcore_axis_name="s", num_cores=NC,
        num_subcores=NS)
    out_type = [jax.ShapeDtypeStruct((NC, N, H), jnp.float32)]
    scratch = [
        pltpu.VMEM((RPT_D, DR * LANES), jnp.int32),
        pltpu.VMEM((RPT_D, DR * LANES), jnp.int32),
        pltpu.VMEM((GROUP, DR * LANES, H), jnp.float32),
        pltpu.VMEM((CHUNK, H), jnp.float32),
        pltpu.VMEM_SHARED((N, H), jnp.float32),
        pltpu.SemaphoreType.DMA((GROUP,)),
        pltpu.SemaphoreType.DMA((GROUP,)),
    ]
    if with_counts:
        out_type.append(jax.ShapeDtypeStruct((NC, N, 16), jnp.float32))
        scratch += [
            pltpu.SemaphoreType.DMA((GROUP,)),
            pltpu.VMEM((DR * LANES, 16), jnp.float32),
            pltpu.VMEM((CHUNK, 16), jnp.float32),
            pltpu.VMEM_SHARED((N, 16), jnp.float32),
        ]
    return pl.kernel(
        functools.partial(_sc_body, with_counts),
        out_type=out_type,
        mesh=mesh,
        scratch_types=scratch,
        compiler_params=pltpu.CompilerParams(use_tc_tiling_on_sc=False),
    )(z, src2d, dst2d)


def kernel(x, edge_index, W1l, b1, W1r, W2l, b2, W2r, W3, b3):
    src = edge_index[0]
    dst = edge_index[1]
    pad = jnp.zeros((ROWS * LANES - E,), jnp.int32)
    src_p = jnp.concatenate([src, pad]).reshape(DROWS, DR * LANES)
    dst_p = jnp.concatenate([dst, pad]).reshape(DROWS, DR * LANES)

    z1, r1 = _tc1(x, W1l, W1r)
    P1, C1 = _sc_agg(z1, src_p, dst_p, with_counts=True)
    h1 = _tc2(P1, C1, r1, b1)
    (P2,) = _sc_agg(h1, src_p, dst_p, with_counts=False)
    return _tc3(P2, C1, h1, W2l, b2, W2r, W3, b3)


# speedup vs baseline: 1.0342x; 1.0342x over previous
import functools

import jax
import jax.numpy as jnp
from jax import lax
from jax.experimental import pallas as pl
from jax.experimental.pallas import tpu as pltpu
from jax.experimental.pallas import tpu_sc as plsc

N = 10000
E = 320000
D = 128
H = 64
OUT = 128

NC = 2
NS = 16
LANES = 128
REAL_ROWS = E // LANES
ROWS = 2560
RPT = ROWS // (NC * NS)
TAIL = REAL_ROWS - RPT * (NC * NS - 1)
RPS = N // NS
CHUNK = 125
GROUP = 5


_R = 2000


def _tc1_body(x_ref, wl_ref, wr_ref, z_ref, r_ref):
    xb = x_ref[...]
    z_ref[...] = jnp.dot(xb, wl_ref[...], preferred_element_type=jnp.float32)
    r_ref[...] = jnp.dot(xb, wr_ref[...], preferred_element_type=jnp.float32)


def _tc1(x, W1l, W1r):
    return pl.pallas_call(
        _tc1_body,
        grid=(N // _R,),
        in_specs=[
            pl.BlockSpec((_R, D), lambda i: (i, 0)),
            pl.BlockSpec((D, H), lambda i: (0, 0)),
            pl.BlockSpec((D, H), lambda i: (0, 0)),
        ],
        out_specs=[
            pl.BlockSpec((_R, H), lambda i: (i, 0)),
            pl.BlockSpec((_R, H), lambda i: (i, 0)),
        ],
        out_shape=[
            jax.ShapeDtypeStruct((N, H), jnp.float32),
            jax.ShapeDtypeStruct((N, H), jnp.float32),
        ],
    )(x, W1l, W1r)


def _elu(v):
    return jnp.where(v > 0, v, jnp.exp(jnp.minimum(v, 0.0)) - 1.0)


def _tc2_body(p_ref, c_ref, r_ref, b_ref, h_ref):
    p = p_ref[...]
    agg = p[0] + p[1]
    c = c_ref[...]
    cnt = c[0, :, 0] + c[1, :, 0]
    inv = 1.0 / jnp.maximum(cnt, 1.0)
    pre = agg * inv[:, None] + b_ref[...] + r_ref[...]
    h_ref[...] = _elu(pre)


def _tc2(P, C, r1, b1):
    return pl.pallas_call(
        _tc2_body,
        grid=(N // _R,),
        in_specs=[
            pl.BlockSpec((NC, _R, H), lambda i: (0, i, 0)),
            pl.BlockSpec((NC, _R, 16), lambda i: (0, i, 0)),
            pl.BlockSpec((_R, H), lambda i: (i, 0)),
            pl.BlockSpec((1, H), lambda i: (0, 0)),
        ],
        out_specs=pl.BlockSpec((_R, H), lambda i: (i, 0)),
        out_shape=jax.ShapeDtypeStruct((N, H), jnp.float32),
    )(P, C, r1, b1.reshape(1, H))


def _tc3_body(p_ref, c_ref, h1_ref, w2l_ref, b2_ref, w2r_ref, w3_ref, b3_ref,
              o_ref):
    p = p_ref[...]
    agg = p[0] + p[1]
    c = c_ref[...]
    cnt = c[0, :, 0] + c[1, :, 0]
    inv = 1.0 / jnp.maximum(cnt, 1.0)
    mean2 = agg * inv[:, None]
    h1 = h1_ref[...]
    pre = (jnp.dot(mean2, w2l_ref[...], preferred_element_type=jnp.float32)
           + b2_ref[...]
           + jnp.dot(h1, w2r_ref[...], preferred_element_type=jnp.float32))
    h2 = _elu(pre)
    o_ref[...] = (jnp.dot(h2, w3_ref[...], preferred_element_type=jnp.float32)
                  + b3_ref[...])


def _tc3(P, C, h1, W2l, b2, W2r, W3, b3):
    return pl.pallas_call(
        _tc3_body,
        grid=(N // _R,),
        in_specs=[
            pl.BlockSpec((NC, _R, H), lambda i: (0, i, 0)),
            pl.BlockSpec((NC, _R, 16), lambda i: (0, i, 0)),
            pl.BlockSpec((_R, H), lambda i: (i, 0)),
            pl.BlockSpec((H, OUT), lambda i: (0, 0)),
            pl.BlockSpec((1, OUT), lambda i: (0, 0)),
            pl.BlockSpec((H, OUT), lambda i: (0, 0)),
            pl.BlockSpec((OUT, OUT), lambda i: (0, 0)),
            pl.BlockSpec((1, OUT), lambda i: (0, 0)),
        ],
        out_specs=pl.BlockSpec((_R, OUT), lambda i: (i, 0)),
        out_shape=jax.ShapeDtypeStruct((N, OUT), jnp.float32),
    )(P, C, h1, W2l, b2.reshape(1, OUT), W2r, W3, b3.reshape(1, OUT))


def _sc_body(with_counts, *refs):
    if with_counts:
        (z_hbm, src_hbm, dst_hbm, out_hbm, cnt_hbm,
         src_v, dst_v, rows_v, zbuf, agg_sh, gsems, ssems, csems,
         ones_v, cbuf, cnt_sh) = refs
    else:
        (z_hbm, src_hbm, dst_hbm, out_hbm,
         src_v, dst_v, rows_v, zbuf, agg_sh, gsems, ssems) = refs

    cid = lax.axis_index("c")
    sid = lax.axis_index("s")
    wid = cid * NS + sid

    zvec = jnp.zeros((16,), jnp.float32)

    def zb(i, carry):
        zbuf[i // 4, pl.ds((i % 4) * 16, 16)] = zvec
        return carry
    lax.fori_loop(0, CHUNK * 4, zb, 0)
    def zcp(k, carry):
        pltpu.sync_copy(zbuf, agg_sh.at[pl.ds(sid * RPS + k * CHUNK, CHUNK)])
        return carry
    lax.fori_loop(0, RPS // CHUNK, zcp, 0)

    if with_counts:
        def cb(i, carry):
            cbuf[i, :] = zvec
            return carry
        lax.fori_loop(0, CHUNK, cb, 0)
        def ccp(k, carry):
            pltpu.sync_copy(cbuf, cnt_sh.at[pl.ds(sid * RPS + k * CHUNK, CHUNK)])
            return carry
        lax.fori_loop(0, RPS // CHUNK, ccp, 0)
        lane = lax.iota(jnp.int32, 16)
        onevec = jnp.where(lane == 0, 1.0, 0.0).astype(jnp.float32)
        def ob(i, carry):
            ones_v[i, :] = onevec
            return carry
        lax.fori_loop(0, LANES, ob, 0)

    pltpu.sync_copy(src_hbm.at[pl.ds(wid * RPT, RPT)], src_v)
    pltpu.sync_copy(dst_hbm.at[pl.ds(wid * RPT, RPT)], dst_v)
    cap = jnp.where(wid == NC * NS - 1, TAIL, RPT)

    plsc.subcore_barrier()

    def make_step(group):
        def step(i, carry):
            j0 = group * i
            gd, sd, cd = [], [], []
            for k in range(group):
                gd.append(pltpu.async_copy(
                    z_hbm.at[src_v.at[j0 + k]], rows_v.at[k], gsems.at[k]))
            for k in range(group):
                gd[k].wait()
                sd.append(pltpu.async_copy(
                    rows_v.at[k], agg_sh.at[dst_v.at[j0 + k]], ssems.at[k],
                    add=True))
                if with_counts:
                    cd.append(pltpu.async_copy(
                        ones_v, cnt_sh.at[dst_v.at[j0 + k]], csems.at[k],
                        add=True))
            for d in sd + cd:
                d.wait()
            return carry
        return step

    @pl.when(wid == NC * NS - 1)
    def _():
        lax.fori_loop(0, TAIL // 4, make_step(4), 0)

    @pl.when(wid != NC * NS - 1)
    def _():
        lax.fori_loop(0, RPT // GROUP, make_step(GROUP), 0)

    plsc.subcore_barrier()

    def ocp(k, carry):
        base = sid * RPS + k * CHUNK
        pltpu.sync_copy(agg_sh.at[pl.ds(base, CHUNK)], zbuf)
        pltpu.sync_copy(zbuf, out_hbm.at[cid, pl.ds(base, CHUNK)])
        if with_counts:
            pltpu.sync_copy(cnt_sh.at[pl.ds(base, CHUNK)], cbuf)
            pltpu.sync_copy(cbuf, cnt_hbm.at[cid, pl.ds(base, CHUNK)])
        return carry
    lax.fori_loop(0, RPS // CHUNK, ocp, 0)


def _sc_agg(z, src2d, dst2d, with_counts):
    mesh = plsc.VectorSubcoreMesh(
        core_axis_name="c", subcore_axis_name="s", num_cores=NC,
        num_subcores=NS)
    out_type = [jax.ShapeDtypeStruct((NC, N, H), jnp.float32)]
    scratch = [
        pltpu.VMEM((RPT, LANES), jnp.int32),
        pltpu.VMEM((RPT, LANES), jnp.int32),
        pltpu.VMEM((GROUP, LANES, H), jnp.float32),
        pltpu.VMEM((CHUNK, H), jnp.float32),
        pltpu.VMEM_SHARED((N, H), jnp.float32),
        pltpu.SemaphoreType.DMA((GROUP,)),
        pltpu.SemaphoreType.DMA((GROUP,)),
    ]
    if with_counts:
        out_type.append(jax.ShapeDtypeStruct((NC, N, 16), jnp.float32))
        scratch += [
            pltpu.SemaphoreType.DMA((GROUP,)),
            pltpu.VMEM((LANES, 16), jnp.float32),
            pltpu.VMEM((CHUNK, 16), jnp.float32),
            pltpu.VMEM_SHARED((N, 16), jnp.float32),
        ]
    return pl.kernel(
        functools.partial(_sc_body, with_counts),
        out_type=out_type,
        mesh=mesh,
        scratch_types=scratch,
        compiler_params=pltpu.CompilerParams(use_tc_tiling_on_sc=False),
    )(z, src2d, dst2d)


def kernel(x, edge_index, W1l, b1, W1r, W2l, b2, W2r, W3, b3):
    src = edge_index[0]
    dst = edge_index[1]
    pad = jnp.zeros((ROWS * LANES - E,), jnp.int32)
    src_p = jnp.concatenate([src, pad]).reshape(ROWS, LANES)
    dst_p = jnp.concatenate([dst, pad]).reshape(ROWS, LANES)

    z1, r1 = _tc1(x, W1l, W1r)
    P1, C1 = _sc_agg(z1, src_p, dst_p, with_counts=True)
    h1 = _tc2(P1, C1, r1, b1)
    (P2,) = _sc_agg(h1, src_p, dst_p, with_counts=False)
    return _tc3(P2, C1, h1, W2l, b2, W2r, W3, b3)
